# baseline (speedup 1.0000x reference)
import functools
import math

import jax
import jax.numpy as jnp
import numpy as np
from jax import lax
from jax.experimental import pallas as pl
from jax.experimental.pallas import tpu as pltpu
from jax.experimental.pallas import tpu_sc as plsc

NC, NS, L = 2, 16, 16
NW = NC * NS
B, T, D = 4, 2048, 768
TCHUNK = T // NW
HC = 8
GROUPS = TCHUNK // HC
GR = B * HC
NBUF = 4


def _pos_encoding(t, d):
    pos = np.arange(t, dtype=np.float32)[:, None]
    div_term = np.exp(
        np.arange(0, d, 2, dtype=np.float32) * (-math.log(10000.0) / d))
    pe = np.zeros((t, d), dtype=np.float32)
    pe[:, 0::2] = np.sin(pos * div_term, dtype=np.float32)
    pe[:, 1::2] = np.cos(pos * div_term, dtype=np.float32)
    return jnp.asarray(pe)


def _sc_embed(x, pe, emb_table):
    mesh = plsc.VectorSubcoreMesh(core_axis_name="c", subcore_axis_name="s")

    @functools.partial(
        pl.kernel,
        out_type=jax.ShapeDtypeStruct((B, T, D), jnp.float32),
        mesh=mesh,
        scratch_types=[
            pltpu.VMEM((B, TCHUNK), jnp.int32),
            pltpu.VMEM((NBUF, HC, D), jnp.float32),
            pltpu.VMEM((NBUF, GR, D), jnp.float32),
            pltpu.SemaphoreType.DMA,
            pltpu.SemaphoreType.DMA,
            pltpu.SemaphoreType.DMA,
            pltpu.SemaphoreType.DMA,
            pltpu.SemaphoreType.DMA,
            pltpu.SemaphoreType.DMA,
            pltpu.SemaphoreType.DMA,
            pltpu.SemaphoreType.DMA,
            pltpu.SemaphoreType.DMA,
        ],
    )
    def run(x_hbm, pe_hbm, table_hbm, out_hbm, idx_v, pe_v, rows_v,
            isem, g0, g1, g2, g3, o0, o1, o2, o3):
        gsems = [g0, g1, g2, g3]
        osems = [o0, o1, o2, o3]
        w = lax.axis_index("s") * NC + lax.axis_index("c")
        t0 = w * TCHUNK

        icps = [pltpu.async_copy(x_hbm.at[b, pl.ds(t0, TCHUNK)],
                                 idx_v.at[b], isem) for b in range(B)]
        for c in icps:
            c.wait()

        def gather(g):
            s = g % NBUF
            cps = [pltpu.async_copy(
                table_hbm.at[idx_v.at[b, pl.ds(g * HC, HC)]],
                rows_v.at[s, pl.ds(b * HC, HC)], gsems[s]) for b in range(B)]
            cps.append(pltpu.async_copy(
                pe_hbm.at[pl.ds(t0 + g * HC, HC)], pe_v.at[s], gsems[s]))
            return cps

        gcps = [None] * GROUPS
        ocps = [[None] * B for _ in range(GROUPS)]
        waited = set()
        for g in range(2):
            gcps[g] = gather(g)

        for g in range(GROUPS):
            tgt = g + 2
            if tgt < GROUPS:
                if tgt >= NBUF:
                    for c in ocps[tgt - NBUF]:
                        c.wait()
                    waited.add(tgt - NBUF)
                gcps[tgt] = gather(tgt)
            s = g % NBUF
            hb = HC * g
            for c in gcps[g]:
                c.wait()

            def row_add(i, _, s=s, hb=hb):
                for j in range(D // L):
                    sl = pl.ds(j * L, L)
                    pv = pe_v[s, i, sl]
                    for b in range(B):
                        plsc.addupdate(rows_v.at[s, b * HC + i, sl], pv)
                return 0

            lax.fori_loop(0, HC, row_add, 0)
            for b in range(B):
                ocps[g][b] = pltpu.async_copy(
                    rows_v.at[s, pl.ds(b * HC, HC)],
                    out_hbm.at[b, pl.ds(t0 + hb, HC)], osems[s])

        for g in range(GROUPS):
            if g not in waited:
                for c in ocps[g]:
                    c.wait()

    return run(x, pe, emb_table)


_PE = _pos_encoding(T, D)


def kernel(x, emb_table):
    return _sc_embed(x.astype(jnp.int32), _PE, emb_table)


# speedup vs baseline: 1.0401x; 1.0401x over previous
import functools
import math

import jax
import jax.numpy as jnp
import numpy as np
from jax import lax
from jax.experimental import pallas as pl
from jax.experimental.pallas import tpu as pltpu
from jax.experimental.pallas import tpu_sc as plsc

NC, NS, L = 2, 16, 16
NW = NC * NS
B, T, D = 4, 2048, 768
TCHUNK = T // NW
HC = 8
GROUPS = TCHUNK // HC
GR = B * HC
NBUF = 3


def _pos_encoding(t, d):
    pos = np.arange(t, dtype=np.float32)[:, None]
    div_term = np.exp(
        np.arange(0, d, 2, dtype=np.float32) * (-math.log(10000.0) / d))
    pe = np.zeros((t, d), dtype=np.float32)
    pe[:, 0::2] = np.sin(pos * div_term, dtype=np.float32)
    pe[:, 1::2] = np.cos(pos * div_term, dtype=np.float32)
    return jnp.asarray(pe)


def _sc_embed(x, pe, emb_table):
    mesh = plsc.VectorSubcoreMesh(core_axis_name="c", subcore_axis_name="s")

    @functools.partial(
        pl.kernel,
        out_type=jax.ShapeDtypeStruct((B, T, D), jnp.float32),
        mesh=mesh,
        scratch_types=[
            pltpu.VMEM((B, TCHUNK), jnp.int32),
            pltpu.VMEM((TCHUNK, D), jnp.float32),
            pltpu.VMEM((NBUF, GR, D), jnp.float32),
            pltpu.SemaphoreType.DMA,
            pltpu.SemaphoreType.DMA,
            pltpu.SemaphoreType.DMA,
            pltpu.SemaphoreType.DMA,
            pltpu.SemaphoreType.DMA,
            pltpu.SemaphoreType.DMA,
            pltpu.SemaphoreType.DMA,
            pltpu.SemaphoreType.DMA,
        ],
    )
    def run(x_hbm, pe_hbm, table_hbm, out_hbm, idx_v, pe_v, rows_v,
            pesem, isem, g0, g1, g2, o0, o1, o2):
        gsems = [g0, g1, g2]
        osems = [o0, o1, o2]
        w = lax.axis_index("s") * NC + lax.axis_index("c")
        t0 = w * TCHUNK

        pe_cp = pltpu.async_copy(pe_hbm.at[pl.ds(t0, TCHUNK)], pe_v, pesem)
        icps = [pltpu.async_copy(x_hbm.at[b, pl.ds(t0, TCHUNK)],
                                 idx_v.at[b], isem) for b in range(B)]
        for c in icps:
            c.wait()

        def gather(g):
            s = g % NBUF
            return [pltpu.async_copy(
                table_hbm.at[idx_v.at[b, pl.ds(g * HC, HC)]],
                rows_v.at[s, pl.ds(b * HC, HC)], gsems[s]) for b in range(B)]

        gcps = [None] * GROUPS
        ocps = [[None] * B for _ in range(GROUPS)]
        waited = set()
        for g in range(2):
            gcps[g] = gather(g)
        pe_cp.wait()

        for g in range(GROUPS):
            tgt = g + 2
            if tgt < GROUPS:
                if tgt >= NBUF:
                    for c in ocps[tgt - NBUF]:
                        c.wait()
                    waited.add(tgt - NBUF)
                gcps[tgt] = gather(tgt)
            s = g % NBUF
            hb = HC * g
            for c in gcps[g]:
                c.wait()

            def row_add(i, _, s=s, hb=hb):
                for j in range(D // L):
                    sl = pl.ds(j * L, L)
                    pv = pe_v[hb + i, sl]
                    for b in range(B):
                        plsc.addupdate(rows_v.at[s, b * HC + i, sl], pv)
                return 0

            lax.fori_loop(0, HC, row_add, 0)
            for b in range(B):
                ocps[g][b] = pltpu.async_copy(
                    rows_v.at[s, pl.ds(b * HC, HC)],
                    out_hbm.at[b, pl.ds(t0 + hb, HC)], osems[s])

        for g in range(GROUPS):
            if g not in waited:
                for c in ocps[g]:
                    c.wait()

    return run(x, pe, emb_table)


_PE = _pos_encoding(T, D)


def kernel(x, emb_table):
    return _sc_embed(x.astype(jnp.int32), _PE, emb_table)


# speedup vs baseline: 1.0412x; 1.0010x over previous
import functools
import math

import jax
import jax.numpy as jnp
import numpy as np
from jax import lax
from jax.experimental import pallas as pl
from jax.experimental.pallas import tpu as pltpu
from jax.experimental.pallas import tpu_sc as plsc

NC, NS, L = 2, 16, 16
NW = NC * NS
B, T, D = 4, 2048, 768
TCHUNK = T // NW
HC = 8
GROUPS = TCHUNK // HC
GR = B * HC
NBUF = 3


def _pos_encoding(t, d):
    pos = np.arange(t, dtype=np.float32)[:, None]
    div_term = np.exp(
        np.arange(0, d, 2, dtype=np.float32) * (-math.log(10000.0) / d))
    pe = np.zeros((t, d), dtype=np.float32)
    pe[:, 0::2] = np.sin(pos * div_term, dtype=np.float32)
    pe[:, 1::2] = np.cos(pos * div_term, dtype=np.float32)
    return pe


def _sc_embed(x, pe, emb_table):
    mesh = plsc.VectorSubcoreMesh(core_axis_name="c", subcore_axis_name="s")

    @functools.partial(
        pl.kernel,
        out_type=jax.ShapeDtypeStruct((B, T, D), jnp.float32),
        mesh=mesh,
        scratch_types=[
            pltpu.VMEM((B, TCHUNK), jnp.int32),
            pltpu.VMEM((TCHUNK, D), jnp.float32),
            pltpu.VMEM((NBUF, GR, D), jnp.float32),
            pltpu.SemaphoreType.DMA,
            pltpu.SemaphoreType.DMA,
            pltpu.SemaphoreType.DMA,
            pltpu.SemaphoreType.DMA,
            pltpu.SemaphoreType.DMA,
            pltpu.SemaphoreType.DMA,
            pltpu.SemaphoreType.DMA,
            pltpu.SemaphoreType.DMA,
        ],
    )
    def run(x_hbm, pe_hbm, table_hbm, out_hbm, idx_v, pe_v, rows_v,
            pesem, isem, g0, g1, g2, o0, o1, o2):
        gsems = [g0, g1, g2]
        osems = [o0, o1, o2]
        w = lax.axis_index("s") * NC + lax.axis_index("c")
        t0 = w * TCHUNK

        pe_cp = pltpu.async_copy(pe_hbm.at[pl.ds(t0, TCHUNK)], pe_v, pesem)
        icps = [pltpu.async_copy(x_hbm.at[b, pl.ds(t0, TCHUNK)],
                                 idx_v.at[b], isem) for b in range(B)]
        for c in icps:
            c.wait()

        def gather(g):
            s = g % NBUF
            return [pltpu.async_copy(
                table_hbm.at[idx_v.at[b, pl.ds(g * HC, HC)]],
                rows_v.at[s, pl.ds(b * HC, HC)], gsems[s]) for b in range(B)]

        gcps = [None] * GROUPS
        ocps = [[None] * B for _ in range(GROUPS)]
        waited = set()
        for g in range(2):
            gcps[g] = gather(g)
        pe_cp.wait()

        for g in range(GROUPS):
            tgt = g + 2
            if tgt < GROUPS:
                if tgt >= NBUF:
                    for c in ocps[tgt - NBUF]:
                        c.wait()
                    waited.add(tgt - NBUF)
                gcps[tgt] = gather(tgt)
            s = g % NBUF
            hb = HC * g
            for c in gcps[g]:
                c.wait()

            def row_add(i, _, s=s, hb=hb):
                for j in range(D // L):
                    sl = pl.ds(j * L, L)
                    pv = pe_v[hb + i, sl]
                    for b in range(B):
                        plsc.addupdate(rows_v.at[s, b * HC + i, sl], pv)
                return 0

            lax.fori_loop(0, HC, row_add, 0)
            for b in range(B):
                ocps[g][b] = pltpu.async_copy(
                    rows_v.at[s, pl.ds(b * HC, HC)],
                    out_hbm.at[b, pl.ds(t0 + hb, HC)], osems[s])

        for g in range(GROUPS):
            if g not in waited:
                for c in ocps[g]:
                    c.wait()

    return run(x, pe, emb_table)


def kernel(x, emb_table):
    pe = jnp.asarray(_pos_encoding(T, D))
    return _sc_embed(x.astype(jnp.int32), pe, emb_table)
